# baseline (speedup 1.0000x reference)
import functools

import jax
import jax.numpy as jnp
from jax import lax
from jax.experimental import pallas as pl
from jax.experimental.pallas import tpu as pltpu
from jax.experimental.pallas import tpu_sc as plsc

_G = 7
_B, _N = 128, 32768
_L = 16
_SEG = _N // _L
_RADIX = 2048
_NCH = 8
_CV = _SEG // _NCH
_NC, _NS = 2, 16
_NW = _NC * _NS
_RPW = _B // _NW
_MSB = -2147483648


@functools.partial(
    pl.kernel,
    mesh=plsc.VectorSubcoreMesh(core_axis_name="c", subcore_axis_name="s"),
    compiler_params=pltpu.CompilerParams(needs_layout_passes=False),
    out_type=(
        jax.ShapeDtypeStruct((_B * _N,), jnp.float32),
        jax.ShapeDtypeStruct((_B * _N,), jnp.int32),
    ),
    scratch_types=[
        pltpu.VMEM((_N,), jnp.float32),
        pltpu.VMEM((_N,), jnp.int32),
        pltpu.VMEM((_N,), jnp.int32),
    ] + [
        pltpu.VMEM((_RADIX,), jnp.int32)
        for _ in range(_NCH)
    ] + [
        pltpu.VMEM((4096,), jnp.float32),
        pltpu.VMEM((_L,), jnp.float32),
        pltpu.VMEM((_L,), jnp.float32),
        pltpu.SemaphoreType.DMA,
    ],
)
def _sc_sort(x_hbm, bc_hbm, idx_hbm, kv, idx, pbuf,
             h0, h1, h2, h3, h4, h5, h6, h7, zbuf, headv, tailv, zsem):
    hists = (h0, h1, h2, h3, h4, h5, h6, h7)
    wid = lax.axis_index("s") * _NC + lax.axis_index("c")
    lane = lax.iota(jnp.int32, _L)
    zero16f = jnp.zeros((_L,), jnp.float32)
    zero16i = jnp.zeros((_L,), jnp.int32)

    def _z(i, c):
        zbuf[pl.ds(i * _L, _L)] = zero16f
        return c
    lax.fori_loop(0, 4096 // _L, _z, 0)

    def _row_body(r, c_row):
        row = wid * _RPW + r
        base = row * _N

        pltpu.sync_copy(x_hbm.at[pl.ds(base, _N)], kv)
        zcopies = []
        for cidx in range(8):
            off = 16 + cidx * 4096
            ln = 4096 if cidx < 7 else 4064
            zcopies.append(pltpu.async_copy(
                zbuf.at[pl.ds(0, ln)],
                bc_hbm.at[pl.ds(base + off, ln)], zsem))

        for p in range(3):
            shift = (0, 11, 21)[p]

            @plsc.parallel_loop(0, _RADIX // _L, unroll=4)
            def _hzero(i):
                for w in range(_NCH):
                    hists[w][pl.ds(i * _L, _L)] = zero16i

            for w in range(_NCH):
                @plsc.parallel_loop(0, _CV, unroll=16)
                def _pa(tt, _w=w):
                    t = _w * _CV + tt
                    if p == 0:
                        v = kv[pl.ds(t * _L, _L)]
                        b = lax.bitcast_convert_type(v, jnp.int32)
                        k = jnp.where(v >= 0.0,
                                      jnp.invert(jnp.bitwise_or(b, _MSB)), b)
                        kv[pl.ds(t * _L, _L)] = \
                            lax.bitcast_convert_type(k, jnp.float32)
                        cur = t * _L + lane
                    else:
                        cur = idx[pl.ds(t * _L, _L)]
                        k = lax.bitcast_convert_type(
                            plsc.load_gather(kv, [cur]), jnp.int32)
                    d = jnp.bitwise_and(
                        jnp.right_shift(k, shift), _RADIX - 1)
                    occ, lastm = plsc.scan_count(d)
                    pw = jnp.bitwise_or(
                        jnp.bitwise_or(cur, jnp.left_shift(d, 15)),
                        jnp.bitwise_or(jnp.left_shift(occ - 1, 26),
                                       jnp.where(lastm, _MSB, 0)))
                    pbuf[pl.ds(t * _L, _L)] = pw
                    plsc.addupdate_scatter(hists[_w], [d], occ, mask=lastm)

            def _scan(i, carry):
                vs = [hists[w][pl.ds(i * _L, _L)] for w in range(_NCH)]
                tot = vs[0]
                for w in range(1, _NCH):
                    tot = tot + vs[w]
                inc = plsc.cumsum(tot)
                run = inc - tot + carry
                for w in range(_NCH):
                    hists[w][pl.ds(i * _L, _L)] = run
                    run = run + vs[w]
                return carry + jnp.sum(tot)
            lax.fori_loop(0, _RADIX // _L, _scan, jnp.int32(0))

            def _pb(th, c):
                for u in range(4):
                    tt = th * 4 + u
                    for w in range(_NCH):
                        pw = pbuf[pl.ds((w * _CV + tt) * _L, _L)]
                        cur = jnp.bitwise_and(pw, 32767)
                        d = jnp.bitwise_and(
                            jnp.right_shift(pw, 15), _RADIX - 1)
                        q = jnp.bitwise_and(jnp.right_shift(pw, 26), 15)
                        lastm = pw < 0
                        base_o = plsc.load_gather(hists[w], [d])
                        plsc.store_scatter(idx, [base_o + q], cur)
                        plsc.store_scatter(hists[w], [d], base_o + q + 1,
                                           mask=lastm)
                return c
            lax.fori_loop(0, _CV // 4, _pb, 0)

        pltpu.sync_copy(idx, idx_hbm.at[pl.ds(base, _N)])

        def _invert_keys(idx16):
            k = lax.bitcast_convert_type(
                plsc.load_gather(kv, [idx16]), jnp.int32)
            bits = jnp.where(k < 0, k, jnp.bitwise_and(jnp.invert(k), ~_MSB))
            return lax.bitcast_convert_type(bits, jnp.float32)

        top = _invert_keys(idx[pl.ds(0, _L)])
        mh = lane < _G
        mt = jnp.where(mh, top, -3e38)
        eh = jnp.where(mh, jnp.exp(mt - jnp.max(mt)), 0.0)
        headv[...] = eh / jnp.sum(eh)

        bot = _invert_keys(idx[pl.ds(_N - _L, _L)])
        tl = 1.0 - bot
        ml = lane >= (_L - _G)
        mtl = jnp.where(ml, tl, -3e38)
        el = jnp.where(ml, jnp.exp(mtl - jnp.max(mtl)), 0.0)
        tailv[...] = -(el / jnp.sum(el))

        for zc in zcopies:
            zc.wait()
        pltpu.sync_copy(headv, bc_hbm.at[pl.ds(base, _L)])
        pltpu.sync_copy(tailv, bc_hbm.at[pl.ds(base + _N - _L, _L)])
        return c_row
    lax.fori_loop(0, _RPW, _row_body, 0)


def kernel(x):
    bc_flat, idx_flat = _sc_sort(x.reshape(-1))
    return (bc_flat.reshape(_B, _N), idx_flat.reshape(_B, _N))


# speedup vs baseline: 1.0507x; 1.0507x over previous
import functools

import jax
import jax.numpy as jnp
from jax import lax
from jax.experimental import pallas as pl
from jax.experimental.pallas import tpu as pltpu
from jax.experimental.pallas import tpu_sc as plsc

_G = 7
_B, _N = 128, 32768
_L = 16
_SEG = _N // _L
_RADIX = 2048
_NCH = 8
_CV = _SEG // _NCH
_NC, _NS = 2, 16
_NW = _NC * _NS
_RPW = _B // _NW
_MSB = -2147483648


@functools.partial(
    pl.kernel,
    mesh=plsc.VectorSubcoreMesh(core_axis_name="c", subcore_axis_name="s"),
    compiler_params=pltpu.CompilerParams(needs_layout_passes=False),
    out_type=(
        jax.ShapeDtypeStruct((_B * _N,), jnp.float32),
        jax.ShapeDtypeStruct((_B * _N,), jnp.int32),
    ),
    scratch_types=[
        pltpu.VMEM((_N,), jnp.float32),
        pltpu.VMEM((_N,), jnp.int32),
        pltpu.VMEM((_N,), jnp.int32),
    ] + [
        pltpu.VMEM((_RADIX,), jnp.int32)
        for _ in range(_NCH)
    ] + [
        pltpu.VMEM((4096,), jnp.float32),
        pltpu.VMEM((_L,), jnp.float32),
        pltpu.VMEM((_L,), jnp.float32),
        pltpu.SemaphoreType.DMA,
    ],
)
def _sc_sort(x_hbm, bc_hbm, idx_hbm, kv, idx, pbuf,
             h0, h1, h2, h3, h4, h5, h6, h7, zbuf, headv, tailv, zsem):
    hists = (h0, h1, h2, h3, h4, h5, h6, h7)
    wid = lax.axis_index("s") * _NC + lax.axis_index("c")
    lane = lax.iota(jnp.int32, _L)
    zero16f = jnp.zeros((_L,), jnp.float32)
    zero16i = jnp.zeros((_L,), jnp.int32)

    def _z(i, c):
        zbuf[pl.ds(i * _L, _L)] = zero16f
        return c
    lax.fori_loop(0, 4096 // _L, _z, 0)

    def _row_body(r, c_row):
        row = wid * _RPW + r
        base = row * _N

        pltpu.sync_copy(x_hbm.at[pl.ds(base, _N)], kv)
        zcopies = []
        for cidx in range(8):
            off = 16 + cidx * 4096
            ln = 4096 if cidx < 7 else 4064
            zcopies.append(pltpu.async_copy(
                zbuf.at[pl.ds(0, ln)],
                bc_hbm.at[pl.ds(base + off, ln)], zsem))

        for p in range(3):
            shift = (0, 11, 21)[p]

            @plsc.parallel_loop(0, _RADIX // _L, unroll=4)
            def _hzero(i):
                for w in range(_NCH):
                    hists[w][pl.ds(i * _L, _L)] = zero16i

            for w in range(_NCH):
                @plsc.parallel_loop(0, _CV, unroll=8)
                def _pa(tt, _w=w):
                    t = _w * _CV + tt
                    if p == 0:
                        v = kv[pl.ds(t * _L, _L)]
                        b = lax.bitcast_convert_type(v, jnp.int32)
                        k = jnp.where(v >= 0.0,
                                      jnp.invert(jnp.bitwise_or(b, _MSB)), b)
                        kv[pl.ds(t * _L, _L)] = \
                            lax.bitcast_convert_type(k, jnp.float32)
                        cur = t * _L + lane
                    else:
                        cur = idx[pl.ds(t * _L, _L)]
                        k = lax.bitcast_convert_type(
                            plsc.load_gather(kv, [cur]), jnp.int32)
                    d = jnp.bitwise_and(
                        jnp.right_shift(k, shift), _RADIX - 1)
                    occ, lastm = plsc.scan_count(d)
                    pw = jnp.bitwise_or(
                        jnp.bitwise_or(cur, jnp.left_shift(d, 15)),
                        jnp.bitwise_or(jnp.left_shift(occ - 1, 26),
                                       jnp.where(lastm, _MSB, 0)))
                    pbuf[pl.ds(t * _L, _L)] = pw
                    plsc.addupdate_scatter(hists[_w], [d], occ, mask=lastm)

            def _scan(i, carry):
                vs = [hists[w][pl.ds(i * _L, _L)] for w in range(_NCH)]
                tot = vs[0]
                for w in range(1, _NCH):
                    tot = tot + vs[w]
                inc = plsc.cumsum(tot)
                run = inc - tot + carry
                for w in range(_NCH):
                    hists[w][pl.ds(i * _L, _L)] = run
                    run = run + vs[w]
                return carry + jnp.sum(tot)
            lax.fori_loop(0, _RADIX // _L, _scan, jnp.int32(0))

            def _pb(th, c):
                for u in range(2):
                    tt = th * 2 + u
                    for w in range(_NCH):
                        pw = pbuf[pl.ds((w * _CV + tt) * _L, _L)]
                        cur = jnp.bitwise_and(pw, 32767)
                        d = jnp.bitwise_and(
                            jnp.right_shift(pw, 15), _RADIX - 1)
                        q = jnp.bitwise_and(jnp.right_shift(pw, 26), 15)
                        lastm = pw < 0
                        base_o = plsc.load_gather(hists[w], [d])
                        plsc.store_scatter(idx, [base_o + q], cur)
                        plsc.store_scatter(hists[w], [d], base_o + q + 1,
                                           mask=lastm)
                return c
            lax.fori_loop(0, _CV // 2, _pb, 0)

        pltpu.sync_copy(idx, idx_hbm.at[pl.ds(base, _N)])

        def _invert_keys(idx16):
            k = lax.bitcast_convert_type(
                plsc.load_gather(kv, [idx16]), jnp.int32)
            bits = jnp.where(k < 0, k, jnp.bitwise_and(jnp.invert(k), ~_MSB))
            return lax.bitcast_convert_type(bits, jnp.float32)

        top = _invert_keys(idx[pl.ds(0, _L)])
        mh = lane < _G
        mt = jnp.where(mh, top, -3e38)
        eh = jnp.where(mh, jnp.exp(mt - jnp.max(mt)), 0.0)
        headv[...] = eh / jnp.sum(eh)

        bot = _invert_keys(idx[pl.ds(_N - _L, _L)])
        tl = 1.0 - bot
        ml = lane >= (_L - _G)
        mtl = jnp.where(ml, tl, -3e38)
        el = jnp.where(ml, jnp.exp(mtl - jnp.max(mtl)), 0.0)
        tailv[...] = -(el / jnp.sum(el))

        for zc in zcopies:
            zc.wait()
        pltpu.sync_copy(headv, bc_hbm.at[pl.ds(base, _L)])
        pltpu.sync_copy(tailv, bc_hbm.at[pl.ds(base + _N - _L, _L)])
        return c_row
    lax.fori_loop(0, _RPW, _row_body, 0)


def kernel(x):
    bc_flat, idx_flat = _sc_sort(x.reshape(-1))
    return (bc_flat.reshape(_B, _N), idx_flat.reshape(_B, _N))
